# baseline (speedup 1.0000x reference)
import functools

import jax
import jax.numpy as jnp
from jax import lax
from jax.experimental import pallas as pl
from jax.experimental.pallas import tpu as pltpu
from jax.experimental.pallas import tpu_sc as plsc

H = 512
W = 512
N_PTS = 65536
D = 256
K = 16
Q = 8192
L = 16
NC = 2
NS = 16
NW = NC * NS
QT = Q // NW
OB = 32

FIX = 64.0

T0 = 6.27930532
T1 = -41.11083325
T2 = 78.05022265
T3 = -56.33605013

INV2PI = 0.15915493667125702
MAGIC = 1.5 * 2 ** 23


def _stage1_body(pxq, pyq, pxy, nm2, nbr_out, w_out,
                 qx_v, qy_v, rq_v, fx_v, fy_v, nbr_v, nbr_f, w_f, pxy_v,
                 semm):
    wid = lax.axis_index("s") * NC + lax.axis_index("c")
    base = wid * QT

    pltpu.sync_copy(pxq.at[pl.ds(base, QT)], qx_v)
    pltpu.sync_copy(pyq.at[pl.ds(base, QT)], qy_v)
    pltpu.sync_copy(pxy, pxy_v)

    @pl.loop(0, QT // L)
    def _stage2(g):
        off = g * L
        qx16 = qx_v[pl.ds(off, L)]
        qy16 = qy_v[pl.ds(off, L)]
        ixi = qx16.astype(jnp.int32)
        iyi = qy16.astype(jnp.int32)
        rq_v[pl.ds(off, L)] = ixi * W + iyi
        fx_v[pl.ds(off, L)] = ixi.astype(jnp.float32)
        fy_v[pl.ds(off, L)] = iyi.astype(jnp.float32)

    half = QT // 2
    c0 = pltpu.async_copy(nm2.at[rq_v.at[pl.ds(0, half)]],
                          nbr_v.at[pl.ds(0, half), :], semm)
    c1 = pltpu.async_copy(nm2.at[rq_v.at[pl.ds(half, half)]],
                          nbr_v.at[pl.ds(half, half), :], semm)
    c0.wait()
    c1.wait()

    @pl.loop(0, QT)
    def _weights(q):
        nv = nbr_v[q, :]
        nbr_f[pl.ds(q * K, K)] = nv
        pk = plsc.load_gather(pxy_v, [nv])
        shift = jnp.full((L,), 16, jnp.int32)
        xk = lax.shift_right_logical(pk, shift).astype(jnp.float32) * jnp.float32(1.0 / FIX)
        yk = (pk & jnp.int32(0xFFFF)).astype(jnp.float32) * jnp.float32(1.0 / FIX)
        dx = xk - fx_v[pl.ds(q, L)][0]
        dy = yk - fy_v[pl.ds(q, L)][0]
        d2 = dx * dx + dy * dy
        seed = plsc.bitcast(
            jnp.int32(0x5F3759DF) - lax.shift_right_logical(
                plsc.bitcast(d2, jnp.int32), jnp.full((L,), 1, jnp.int32)),
            jnp.float32)
        hx = d2 * jnp.float32(0.5)
        y = seed
        y = y * (jnp.float32(1.5) - hx * y * y)
        y = y * (jnp.float32(1.5) - hx * y * y)
        y = y * (jnp.float32(1.5) - hx * y * y)
        dist = d2 * y
        total = jnp.sum(dist)
        w_f[pl.ds(q * K, K)] = jnp.float32(1.0) - dist / (
            total + jnp.full((L,), 1e-8, jnp.float32))

    pltpu.sync_copy(nbr_f, nbr_out.at[pl.ds(base * K, QT * K)])
    pltpu.sync_copy(w_f, w_out.at[pl.ds(base * K, QT * K)])


def _stage2_body(nbrs, wts, emb, harm, out,
                 nbr_v, w_v, h2i_v, e0, e1, obuf, sem0, sem1):
    wid = lax.axis_index("s") * NC + lax.axis_index("c")
    base = wid * QT

    pltpu.sync_copy(nbrs.at[pl.ds(base * K, QT * K)], nbr_v)
    pltpu.sync_copy(wts.at[pl.ds(base * K, QT * K)],
                    w_v.at[pl.ds(0, QT * K)])
    pltpu.sync_copy(harm, h2i_v)

    @pl.loop(0, D // L)
    def _scale_h(j):
        off = j * L
        h2i_v[pl.ds(off, L)] = h2i_v[pl.ds(off, L)] * jnp.float32(INV2PI)

    def fire(qi, ebuf, sem):
        pltpu.async_copy(emb.at[nbr_v.at[pl.ds(qi * K, K)]], ebuf, sem)

    def wait(qi, ebuf, sem):
        pltpu.make_async_copy(emb.at[nbr_v.at[pl.ds(qi * K, K)]], ebuf,
                              sem).wait()

    fire(0, e0, sem0)

    def _sin_turns(u):
        nf = (u + jnp.float32(MAGIC)) - jnp.float32(MAGIC)
        t = u - nf
        s = t * t
        p = jnp.float32(T3)
        p = p * s + jnp.float32(T2)
        p = p * s + jnp.float32(T1)
        p = p * s + jnp.float32(T0)
        return t * p

    def process(q, ebuf):
        qq = q % OB
        wbase = q * K

        @pl.loop(0, D // (2 * L))
        def _jloop(j):
            joff = j * (2 * L)
            h2a = h2i_v[pl.ds(joff, L)]
            h2b = h2i_v[pl.ds(joff + L, L)]
            zero = jnp.zeros((L,), jnp.float32)

            @pl.loop(0, K, init_carry=(zero, zero), unroll=K)
            def _kloop(k, accs):
                acca, accb = accs
                wk = w_v[pl.ds(wbase + k, L)][0]
                ea = ebuf[k, pl.ds(joff, L)]
                eb = ebuf[k, pl.ds(joff + L, L)]
                return (acca + _sin_turns(h2a * wk) * ea,
                        accb + _sin_turns(h2b * wk) * eb)

            acca, accb = _kloop
            obuf[qq, pl.ds(joff, L)] = acca
            obuf[qq, pl.ds(joff + L, L)] = accb

        @pl.when(qq == OB - 1)
        def _flush():
            row0 = pl.multiple_of(base + q - (OB - 1), OB)
            pltpu.sync_copy(obuf, out.at[pl.ds(row0, OB), :])

    @pl.loop(0, QT, step=2)
    def _main(q2):
        for b in range(2):
            q = q2 + b
            ebuf = e0 if b == 0 else e1
            sem = sem0 if b == 0 else sem1
            nxt = q + 1

            @pl.when(nxt < QT)
            def _prefetch():
                fire(nxt, e1 if b == 0 else e0, sem1 if b == 0 else sem0)

            wait(q, ebuf, sem)
            process(q, ebuf)


@functools.partial(jax.jit, static_argnames=())
def _latent_map_sc(pxq, pyq, pxy, emb, harm, nm2):
    mesh = plsc.VectorSubcoreMesh(core_axis_name="c", subcore_axis_name="s")
    nbrs, wts = pl.kernel(
        _stage1_body,
        out_type=(jax.ShapeDtypeStruct((Q * K,), jnp.int32),
                  jax.ShapeDtypeStruct((Q * K,), jnp.float32)),
        mesh=mesh,
        compiler_params=pltpu.CompilerParams(
            needs_layout_passes=False, use_tc_tiling_on_sc=False),
        scratch_types=[
            pltpu.VMEM((QT,), jnp.float32),
            pltpu.VMEM((QT,), jnp.float32),
            pltpu.VMEM((QT,), jnp.int32),
            pltpu.VMEM((QT + L,), jnp.float32),
            pltpu.VMEM((QT + L,), jnp.float32),
            pltpu.VMEM((QT, K), jnp.int32),
            pltpu.VMEM((QT * K,), jnp.int32),
            pltpu.VMEM((QT * K,), jnp.float32),
            pltpu.VMEM((N_PTS,), jnp.int32),
            pltpu.SemaphoreType.DMA,
        ],
    )(pxq, pyq, pxy, nm2)

    return pl.kernel(
        _stage2_body,
        out_type=jax.ShapeDtypeStruct((Q, D), jnp.float32),
        mesh=mesh,
        compiler_params=pltpu.CompilerParams(
            needs_layout_passes=False, use_tc_tiling_on_sc=True),
        scratch_types=[
            pltpu.VMEM((QT * K,), jnp.int32),
            pltpu.VMEM((QT * K + L,), jnp.float32),
            pltpu.VMEM((D,), jnp.float32),
            pltpu.VMEM((K, D), jnp.float32),
            pltpu.VMEM((K, D), jnp.float32),
            pltpu.VMEM((OB, D), jnp.float32),
            pltpu.SemaphoreType.DMA,
            pltpu.SemaphoreType.DMA,
        ],
    )(nbrs, wts, emb, harm)


def kernel(position, positions, embeddings, harmonics, neighbor_map):
    pxq = position[:, 0]
    pyq = position[:, 1]
    xq = jnp.round(positions[:, 0] * FIX).astype(jnp.int32)
    yq = jnp.round(positions[:, 1] * FIX).astype(jnp.int32)
    pxy = (xq << 16) | yq
    nm2 = neighbor_map.reshape(H * W, K)
    return _latent_map_sc(pxq, pyq, pxy, embeddings, harmonics, nm2)


# speedup vs baseline: 1.4363x; 1.4363x over previous
import functools

import jax
import jax.numpy as jnp
from jax import lax
from jax.experimental import pallas as pl
from jax.experimental.pallas import tpu as pltpu
from jax.experimental.pallas import tpu_sc as plsc

H = 512
W = 512
N_PTS = 65536
D = 256
K = 16
Q = 8192
L = 16
NC = 2
NS = 16
NW = NC * NS
QT = Q // NW
OB = 32
JC = 4

FIX = 64.0

T0 = 6.27930532
T1 = -41.11083325
T2 = 78.05022265
T3 = -56.33605013

INV2PI = 0.15915493667125702
MAGIC = 1.5 * 2 ** 23


def _sc_body(pxq, pyq, pxy, nmf, emb, harm, out,
             qx_v, qy_v, rq_v, fx_v, fy_v, idx_f, nbr_f, w_v, pxy_v, h2i_v,
             e0, e1, obuf, semm, sem0, sem1):
    wid = lax.axis_index("s") * NC + lax.axis_index("c")
    base = wid * QT

    pltpu.sync_copy(pxq.at[pl.ds(base, QT)], qx_v)
    pltpu.sync_copy(pyq.at[pl.ds(base, QT)], qy_v)
    pltpu.sync_copy(pxy, pxy_v)
    pltpu.sync_copy(harm, h2i_v)

    @pl.loop(0, D // L)
    def _scale_h(j):
        off = j * L
        h2i_v[pl.ds(off, L)] = h2i_v[pl.ds(off, L)] * jnp.float32(INV2PI)

    @pl.loop(0, QT // L)
    def _flatten(g):
        off = g * L
        qx16 = qx_v[pl.ds(off, L)]
        qy16 = qy_v[pl.ds(off, L)]
        ixi = qx16.astype(jnp.int32)
        iyi = qy16.astype(jnp.int32)
        sh7 = jnp.full((L,), 7, jnp.int32)
        rq_v[pl.ds(off, L)] = (ixi * (K * W)
                               + lax.shift_right_logical(iyi, sh7) * jnp.int32(8 * 128)
                               + (iyi & jnp.int32(127)))
        fx_v[pl.ds(off, L)] = ixi.astype(jnp.float32)
        fy_v[pl.ds(off, L)] = iyi.astype(jnp.float32)

    kio = lax.iota(jnp.int32, L)
    sh3v = jnp.full((L,), 3, jnp.int32)
    ks = (lax.shift_right_logical(kio, sh3v) * jnp.int32(8 * W)
          + (kio & jnp.int32(7)) * jnp.int32(128))

    @pl.loop(0, QT)
    def _mkidx(q):
        idx_f[pl.ds(q * K, K)] = ks + rq_v[pl.ds(q, L)][0]

    nch = (QT * K) // 128
    for c in range(nch):
        pltpu.async_copy(nmf.at[idx_f.at[pl.ds(c * 128, 128)]],
                         nbr_f.at[pl.ds(c * 128, 128)], semm)
    for c in range(nch):
        pltpu.make_async_copy(nmf.at[idx_f.at[pl.ds(c * 128, 128)]],
                              nbr_f.at[pl.ds(c * 128, 128)], semm).wait()

    @pl.loop(0, QT)
    def _weights(q):
        nv = nbr_f[pl.ds(q * K, K)]
        pk = plsc.load_gather(pxy_v, [nv])
        shift = jnp.full((L,), 16, jnp.int32)
        xk = lax.shift_right_logical(pk, shift).astype(jnp.float32) * jnp.float32(1.0 / FIX)
        yk = (pk & jnp.int32(0xFFFF)).astype(jnp.float32) * jnp.float32(1.0 / FIX)
        dx = xk - fx_v[pl.ds(q, L)][0]
        dy = yk - fy_v[pl.ds(q, L)][0]
        d2 = dx * dx + dy * dy
        seed = plsc.bitcast(
            jnp.int32(0x5F3759DF) - lax.shift_right_logical(
                plsc.bitcast(d2, jnp.int32), jnp.full((L,), 1, jnp.int32)),
            jnp.float32)
        hx = d2 * jnp.float32(0.5)
        y = seed
        y = y * (jnp.float32(1.5) - hx * y * y)
        y = y * (jnp.float32(1.5) - hx * y * y)
        y = y * (jnp.float32(1.5) - hx * y * y)
        dist = d2 * y
        total = jnp.sum(dist)
        w_v[pl.ds(q * K, K)] = jnp.float32(1.0) - dist / (
            total + jnp.full((L,), 1e-8, jnp.float32))

    def fire(qi, ebuf, sem):
        pltpu.async_copy(emb.at[nbr_f.at[pl.ds(qi * K, K)]], ebuf, sem)

    def wait(qi, ebuf, sem):
        pltpu.make_async_copy(emb.at[nbr_f.at[pl.ds(qi * K, K)]], ebuf,
                              sem).wait()

    fire(0, e0, sem0)

    def _sin_turns(u):
        nf = (u + jnp.float32(MAGIC)) - jnp.float32(MAGIC)
        t = u - nf
        s = t * t
        p = jnp.float32(T3)
        p = p * s + jnp.float32(T2)
        p = p * s + jnp.float32(T1)
        p = p * s + jnp.float32(T0)
        return t * p

    def process(q, ebuf):
        qq = q % OB
        wbase = q * K

        @pl.loop(0, D // (JC * L))
        def _jloop(j):
            joff = j * (JC * L)
            hs = [h2i_v[pl.ds(joff + c * L, L)] for c in range(JC)]
            zero = jnp.zeros((L,), jnp.float32)

            @pl.loop(0, K, init_carry=(zero,) * JC, unroll=K)
            def _kloop(k, accs):
                wk = w_v[pl.ds(wbase + k, L)][0]
                return tuple(
                    accs[c] + _sin_turns(hs[c] * wk)
                    * ebuf[k, pl.ds(joff + c * L, L)]
                    for c in range(JC))

            for c in range(JC):
                obuf[qq, pl.ds(joff + c * L, L)] = _kloop[c]

        @pl.when(qq == OB - 1)
        def _flush():
            row0 = pl.multiple_of(base + q - (OB - 1), OB)
            pltpu.sync_copy(obuf, out.at[pl.ds(row0, OB), :])

    @pl.loop(0, QT, step=2)
    def _main(q2):
        for b in range(2):
            q = q2 + b
            ebuf = e0 if b == 0 else e1
            sem = sem0 if b == 0 else sem1
            nxt = q + 1

            @pl.when(nxt < QT)
            def _prefetch():
                fire(nxt, e1 if b == 0 else e0, sem1 if b == 0 else sem0)

            wait(q, ebuf, sem)
            process(q, ebuf)


@functools.partial(jax.jit, static_argnames=())
def _latent_map_sc(pxq, pyq, pxy, emb, harm, nmf):
    mesh = plsc.VectorSubcoreMesh(core_axis_name="c", subcore_axis_name="s")
    return pl.kernel(
        _sc_body,
        out_type=jax.ShapeDtypeStruct((Q, D), jnp.float32),
        mesh=mesh,
        compiler_params=pltpu.CompilerParams(
            needs_layout_passes=False, use_tc_tiling_on_sc=True),
        scratch_types=[
            pltpu.VMEM((QT,), jnp.float32),
            pltpu.VMEM((QT,), jnp.float32),
            pltpu.VMEM((QT + L,), jnp.int32),
            pltpu.VMEM((QT + L,), jnp.float32),
            pltpu.VMEM((QT + L,), jnp.float32),
            pltpu.VMEM((QT * K,), jnp.int32),
            pltpu.VMEM((QT * K,), jnp.int32),
            pltpu.VMEM((QT * K + L,), jnp.float32),
            pltpu.VMEM((N_PTS,), jnp.int32),
            pltpu.VMEM((D,), jnp.float32),
            pltpu.VMEM((K, D), jnp.float32),
            pltpu.VMEM((K, D), jnp.float32),
            pltpu.VMEM((OB, D), jnp.float32),
            pltpu.SemaphoreType.DMA,
            pltpu.SemaphoreType.DMA,
            pltpu.SemaphoreType.DMA,
        ],
    )(pxq, pyq, pxy, nmf, emb, harm)


def kernel(position, positions, embeddings, harmonics, neighbor_map):
    pxq = position[:, 0]
    pyq = position[:, 1]
    xq = jnp.round(positions[:, 0] * FIX).astype(jnp.int32)
    yq = jnp.round(positions[:, 1] * FIX).astype(jnp.int32)
    pxy = (xq << 16) | yq
    nmf = (neighbor_map.transpose(0, 2, 1)
           .reshape(H, K // 8, 8, W // 128, 128)
           .transpose(0, 1, 3, 2, 4).reshape(-1))
    return _latent_map_sc(pxq, pyq, pxy, embeddings, harmonics, nmf)
